# baseline (speedup 1.0000x reference)
import functools

import jax
import jax.numpy as jnp
from jax import lax
from jax.experimental import pallas as pl
from jax.experimental.pallas import tpu as pltpu
from jax.experimental.pallas import tpu_sc as plsc

_CHUNK = 256


@functools.lru_cache(maxsize=None)
def _build(N, M4, D):
    info = plsc.get_sparse_core_info()
    L = info.num_lanes
    num_workers = info.num_cores * info.num_subcores
    rows_per_w = N // num_workers
    assert rows_per_w * num_workers == N
    C = min(_CHUNK, rows_per_w)
    n_chunks = rows_per_w // C
    assert n_chunks * C == rows_per_w

    mesh = plsc.VectorSubcoreMesh(core_axis_name="c", subcore_axis_name="s")

    @functools.partial(
        pl.kernel,
        mesh=mesh,
        out_type=jax.ShapeDtypeStruct((N, D), jnp.float32),
        scratch_types=[
            pltpu.VMEM((C,), jnp.int32),
            pltpu.VMEM((C,), jnp.int32),
            pltpu.VMEM((C, 4 * D), jnp.float32),
            pltpu.VMEM((C, D), jnp.float32),
            pltpu.SemaphoreType.DMA,
        ],
    )
    def gather_kernel(tbl_hbm, ids_hbm, out_hbm, idx_v, rem_v, rows_v, out_v, sem):
        wid = lax.axis_index("s") * info.num_cores + lax.axis_index("c")
        base = wid * rows_per_w

        def chunk_body(j, carry):
            off = base + j * C
            pltpu.sync_copy(ids_hbm.at[pl.ds(off, C)], idx_v)

            def shift_body(t, carry):
                pid = idx_v[pl.ds(t * L, L)]
                rem_v[pl.ds(t * L, L)] = (pid & 3) * D
                idx_v[pl.ds(t * L, L)] = pid >> 2
                return carry

            lax.fori_loop(0, C // L, shift_body, 0, unroll=4)
            pltpu.async_copy(tbl_hbm.at[idx_v], rows_v, sem).wait()

            def sel_body(t, carry):
                rem16 = rem_v[pl.ds(t * L, L)]
                for l in range(L):
                    i = t * L + l
                    rem = rem16[l]
                    out_v[i, pl.ds(0, L)] = rows_v[i, pl.ds(rem, L)]
                    out_v[i, pl.ds(L, L)] = rows_v[i, pl.ds(rem + L, L)]
                return carry

            lax.fori_loop(0, C // L, sel_body, 0)

            pltpu.sync_copy(out_v, out_hbm.at[pl.ds(off, C)])
            return carry

        lax.fori_loop(0, n_chunks, chunk_body, 0)

    return gather_kernel


def kernel(features, point_ids):
    M, D = features.shape
    N = point_ids.shape[0]
    table128 = features.reshape(M * D // 128, 128)
    return _build(N, M * D // 128, D)(table128, point_ids)


# speedup vs baseline: 1.0900x; 1.0900x over previous
import functools

import jax
import jax.numpy as jnp
from jax import lax
from jax.experimental import pallas as pl
from jax.experimental.pallas import tpu as pltpu
from jax.experimental.pallas import tpu_sc as plsc

_CHUNK = 256


@functools.lru_cache(maxsize=None)
def _build(N, M4, D):
    info = plsc.get_sparse_core_info()
    L = info.num_lanes
    num_workers = info.num_cores * info.num_subcores
    rows_per_w = N // num_workers
    assert rows_per_w * num_workers == N
    C = min(_CHUNK, rows_per_w)
    n_chunks = rows_per_w // C
    assert n_chunks * C == rows_per_w

    mesh = plsc.VectorSubcoreMesh(core_axis_name="c", subcore_axis_name="s")

    @functools.partial(
        pl.kernel,
        mesh=mesh,
        out_type=jax.ShapeDtypeStruct((N, D), jnp.float32),
        scratch_types=[
            pltpu.VMEM((C,), jnp.int32),
            pltpu.VMEM((C,), jnp.int32),
            pltpu.VMEM((C, 4 * D), jnp.float32),
            pltpu.VMEM((C, D), jnp.float32),
            pltpu.SemaphoreType.DMA,
        ],
    )
    def gather_kernel(tbl_hbm, ids_hbm, out_hbm, idx_v, rem_v, rows_v, out_v, sem):
        wid = lax.axis_index("s") * info.num_cores + lax.axis_index("c")
        base = wid * rows_per_w

        def chunk_body(j, carry):
            off = base + j * C
            pltpu.sync_copy(ids_hbm.at[pl.ds(off, C)], idx_v)

            def shift_body(t, carry):
                pid = idx_v[pl.ds(t * L, L)]
                rem_v[pl.ds(t * L, L)] = (pid & 3) * D
                idx_v[pl.ds(t * L, L)] = pid >> 2
                return carry

            lax.fori_loop(0, C // L, shift_body, 0, unroll=4)
            pltpu.async_copy(tbl_hbm.at[idx_v], rows_v, sem).wait()

            def sel_body(t, carry):
                rem16 = rem_v[pl.ds(t * L, L)]
                for l in range(L):
                    i = t * L + l
                    rem = rem16[l]
                    out_v[i, pl.ds(0, L)] = rows_v[i, pl.ds(rem, L)]
                    out_v[i, pl.ds(L, L)] = rows_v[i, pl.ds(rem + L, L)]
                return carry

            lax.fori_loop(0, C // L, sel_body, 0)

            pltpu.sync_copy(out_v, out_hbm.at[pl.ds(off, C)])
            return carry

        lax.fori_loop(0, n_chunks, chunk_body, 0)

    return gather_kernel


def kernel(features, point_ids):
    M, D = features.shape
    N = point_ids.shape[0]
    table128 = features.reshape(M * D // 128, 128)
    out = _build(N, M * D // 128, D)(table128, point_ids)
    return jax.lax.optimization_barrier(out)


# speedup vs baseline: 1.3658x; 1.2530x over previous
import functools

import jax
import jax.numpy as jnp
from jax import lax
from jax.experimental import pallas as pl
from jax.experimental.pallas import tpu as pltpu
from jax.experimental.pallas import tpu_sc as plsc

_CHUNK = 512


@functools.lru_cache(maxsize=None)
def _build(N, M, D):
    info = plsc.get_sparse_core_info()
    num_workers = info.num_cores * info.num_subcores
    rows_per_w = N // num_workers
    assert rows_per_w * num_workers == N
    C = min(_CHUNK, rows_per_w)
    n_chunks = rows_per_w // C
    assert n_chunks * C == rows_per_w

    mesh = plsc.VectorSubcoreMesh(core_axis_name="c", subcore_axis_name="s")

    @functools.partial(
        pl.kernel,
        mesh=mesh,
        out_type=jax.ShapeDtypeStruct((N, D), jnp.float32),
        scratch_types=[
            pltpu.VMEM((C,), jnp.int32),
            pltpu.VMEM((C, D), jnp.float32),
            pltpu.SemaphoreType.DMA,
        ],
        compiler_params=pltpu.CompilerParams(use_tc_tiling_on_sc=False),
    )
    def gather_kernel(tbl_hbm, ids_hbm, out_hbm, idx_v, rows_v, sem):
        wid = lax.axis_index("s") * info.num_cores + lax.axis_index("c")
        base = wid * rows_per_w

        def chunk_body(j, carry):
            off = base + j * C
            pltpu.sync_copy(ids_hbm.at[pl.ds(off, C)], idx_v)
            pltpu.async_copy(tbl_hbm.at[idx_v], rows_v, sem).wait()
            pltpu.sync_copy(rows_v, out_hbm.at[pl.ds(off, C)])
            return carry

        lax.fori_loop(0, n_chunks, chunk_body, 0)

    return gather_kernel


def kernel(features, point_ids):
    M, D = features.shape
    N = point_ids.shape[0]
    out = _build(N, M, D)(features, point_ids)
    return jax.lax.optimization_barrier(out)
